# baseline (speedup 1.0000x reference)
import jax
import jax.numpy as jnp
from jax import lax
from jax.experimental import pallas as pl
from jax.experimental.pallas import tpu as pltpu

TOPK = 32
BR = 256
EPS = 1e-8
PRE_ITERS = 10
MAX_ITERS = 32


def _body(xr_ref, xa_ref, w_ref, b_ref, o_ref, xn_ref):
    j = pl.program_id(1)

    @pl.when(j == 0)
    def _():
        xa = xa_ref[0]
        norm = jnp.sqrt(jnp.sum(xa * xa, axis=1, keepdims=True))
        xn_ref[...] = xa / jnp.maximum(norm, EPS)

    xr = xr_ref[0]
    rnorm = jnp.sqrt(jnp.sum(xr * xr, axis=1, keepdims=True))
    xnr = xr / jnp.maximum(rnorm, EPS)

    sim = lax.dot_general(xnr, xn_ref[...],
                          (((1,), (1,)), ((), ())),
                          preferred_element_type=jnp.float32)

    cmax = jnp.max(jnp.reshape(sim, (BR, TOPK, sim.shape[1] // TOPK)), axis=2)
    hi0 = jnp.max(cmax, axis=1, keepdims=True)
    lo0 = jnp.min(cmax, axis=1, keepdims=True) - 3e-7
    done0 = jnp.zeros((BR, 1), dtype=jnp.float32)
    thr0 = lo0
    ones = jnp.ones((sim.shape[1], 8), dtype=jnp.float32)

    def count_gt(t):
        ind = jnp.where(sim > t, 1.0, 0.0)
        c = lax.dot_general(ind, ones, (((1,), (0,)), ((), ())),
                            preferred_element_type=jnp.float32)
        return c[:, 0:1]

    def prestep(_, carry):
        lo, hi = carry
        t = 0.5 * (lo + hi)
        cnt = count_gt(t)
        pred = cnt >= TOPK
        return jnp.where(pred, t, lo), jnp.where(pred, hi, t)

    lo0, hi0 = lax.fori_loop(0, PRE_ITERS, prestep, (lo0, hi0))

    def cond(carry):
        i, _, _, _, done = carry
        return jnp.logical_and(i < MAX_ITERS, jnp.min(done) < 0.5)

    def step(carry):
        i, lo, hi, thr, done = carry
        t = 0.5 * (lo + hi)
        cnt = count_gt(t)
        live = done < 0.5
        eq = jnp.logical_and(cnt == TOPK, live)
        thr = jnp.where(eq, t, thr)
        done = jnp.where(eq, 1.0, done)
        lo = jnp.where(jnp.logical_and(live, cnt >= TOPK), t, lo)
        hi = jnp.where(jnp.logical_and(live, cnt < TOPK), t, hi)
        return i + 1, lo, hi, thr, done

    _, lo, _, thr, done = lax.while_loop(
        cond, step, (jnp.int32(0), lo0, hi0, thr0, done0))
    thr = jnp.where(done > 0.5, thr, lo)

    masked = jnp.where(sim > thr, sim, 0.0)
    agg = lax.dot_general(masked, xa_ref[0],
                          (((1,), (0,)), ((), ())),
                          preferred_element_type=jnp.float32)

    o_ref[0] = lax.dot_general(agg, w_ref[...],
                               (((1,), (0,)), ((), ())),
                               preferred_element_type=jnp.float32) + b_ref[...]


def kernel(x, W, b):
    B, N, L = x.shape
    b2 = jnp.reshape(b, (1, L))
    grid = (B, N // BR)

    out = pl.pallas_call(
        _body,
        grid=grid,
        in_specs=[
            pl.BlockSpec((1, BR, L), lambda bb, j: (bb, j, 0)),
            pl.BlockSpec((1, N, L), lambda bb, j: (bb, 0, 0)),
            pl.BlockSpec((L, L), lambda bb, j: (0, 0)),
            pl.BlockSpec((1, L), lambda bb, j: (0, 0)),
        ],
        out_specs=pl.BlockSpec((1, BR, L), lambda bb, j: (bb, j, 0)),
        out_shape=jax.ShapeDtypeStruct((B, N, L), jnp.float32),
        scratch_shapes=[pltpu.VMEM((N, L), jnp.float32)],
        compiler_params=pltpu.CompilerParams(
            dimension_semantics=("arbitrary", "arbitrary"),
        ),
    )(x, x, W, b2)

    return (out, out)


# speedup vs baseline: 1.3203x; 1.3203x over previous
import jax
import jax.numpy as jnp
from jax import lax
from jax.experimental import pallas as pl
from jax.experimental.pallas import tpu as pltpu

TOPK = 32
BR = 256
EPS = 1e-8
PRE_ITERS = 10
MAX_ITERS = 32


def _body(xr_ref, xa_ref, w_ref, b_ref, o_ref, xn_ref):
    j = pl.program_id(1)

    @pl.when(j == 0)
    def _():
        xa = xa_ref[0]
        norm = jnp.sqrt(jnp.sum(xa * xa, axis=1, keepdims=True))
        xn_ref[...] = xa / jnp.maximum(norm, EPS)

    xr = xr_ref[0]
    rnorm = jnp.sqrt(jnp.sum(xr * xr, axis=1, keepdims=True))
    xnr = xr / jnp.maximum(rnorm, EPS)

    sim = lax.dot_general(xnr, xn_ref[...],
                          (((1,), (1,)), ((), ())),
                          preferred_element_type=jnp.float32)

    cmax = jnp.max(jnp.reshape(sim, (BR, TOPK, sim.shape[1] // TOPK)), axis=2)
    hi0 = jnp.max(cmax, axis=1, keepdims=True)
    lo0 = jnp.min(cmax, axis=1, keepdims=True) - 3e-7
    done0 = jnp.zeros((BR, 1), dtype=jnp.float32)
    thr0 = lo0
    def count_gt(t):
        return jnp.sum((sim > t).astype(jnp.float32), axis=1, keepdims=True)

    def prestep(_, carry):
        lo, hi = carry
        t = 0.5 * (lo + hi)
        cnt = count_gt(t)
        pred = cnt >= TOPK
        return jnp.where(pred, t, lo), jnp.where(pred, hi, t)

    lo0, hi0 = lax.fori_loop(0, PRE_ITERS, prestep, (lo0, hi0))

    def cond(carry):
        i, _, _, _, done = carry
        return jnp.logical_and(i < MAX_ITERS, jnp.min(done) < 0.5)

    def step(carry):
        i, lo, hi, thr, done = carry
        t = 0.5 * (lo + hi)
        cnt = count_gt(t)
        live = done < 0.5
        eq = jnp.logical_and(cnt == TOPK, live)
        thr = jnp.where(eq, t, thr)
        done = jnp.where(eq, 1.0, done)
        lo = jnp.where(jnp.logical_and(live, cnt >= TOPK), t, lo)
        hi = jnp.where(jnp.logical_and(live, cnt < TOPK), t, hi)
        return i + 1, lo, hi, thr, done

    _, lo, _, thr, done = lax.while_loop(
        cond, step, (jnp.int32(0), lo0, hi0, thr0, done0))
    thr = jnp.where(done > 0.5, thr, lo)

    masked = jnp.where(sim > thr, sim, 0.0)
    agg = lax.dot_general(masked, xa_ref[0],
                          (((1,), (0,)), ((), ())),
                          preferred_element_type=jnp.float32)

    o_ref[0] = lax.dot_general(agg, w_ref[...],
                               (((1,), (0,)), ((), ())),
                               preferred_element_type=jnp.float32) + b_ref[...]


def kernel(x, W, b):
    B, N, L = x.shape
    b2 = jnp.reshape(b, (1, L))
    grid = (B, N // BR)

    out = pl.pallas_call(
        _body,
        grid=grid,
        in_specs=[
            pl.BlockSpec((1, BR, L), lambda bb, j: (bb, j, 0)),
            pl.BlockSpec((1, N, L), lambda bb, j: (bb, 0, 0)),
            pl.BlockSpec((L, L), lambda bb, j: (0, 0)),
            pl.BlockSpec((1, L), lambda bb, j: (0, 0)),
        ],
        out_specs=pl.BlockSpec((1, BR, L), lambda bb, j: (bb, j, 0)),
        out_shape=jax.ShapeDtypeStruct((B, N, L), jnp.float32),
        scratch_shapes=[pltpu.VMEM((N, L), jnp.float32)],
        compiler_params=pltpu.CompilerParams(
            dimension_semantics=("arbitrary", "arbitrary"),
        ),
    )(x, x, W, b2)

    return (out, out)


# speedup vs baseline: 1.4386x; 1.0896x over previous
import jax
import jax.numpy as jnp
from jax import lax
from jax.experimental import pallas as pl
from jax.experimental.pallas import tpu as pltpu

TOPK = 32
BR = 256
EPS = 1e-8
MAX_ITERS = 32


def _body(xr_ref, xa_ref, w_ref, b_ref, o_ref, xn_ref):
    j = pl.program_id(1)

    @pl.when(j == 0)
    def _():
        xa = xa_ref[0]
        norm = jnp.sqrt(jnp.sum(xa * xa, axis=1, keepdims=True))
        xn_ref[...] = xa / jnp.maximum(norm, EPS)

    xr = xr_ref[0]
    rnorm = jnp.sqrt(jnp.sum(xr * xr, axis=1, keepdims=True))
    xnr = xr / jnp.maximum(rnorm, EPS)

    sim = lax.dot_general(xnr, xn_ref[...],
                          (((1,), (1,)), ((), ())),
                          preferred_element_type=jnp.float32)

    lo0 = jnp.full((BR, 1), -1.01, dtype=jnp.float32)
    hi0 = jnp.full((BR, 1), 1.01, dtype=jnp.float32)
    done0 = jnp.zeros((BR, 1), dtype=jnp.float32)
    thr0 = lo0
    def count_gt(t):
        return jnp.sum((sim > t).astype(jnp.float32), axis=1, keepdims=True)

    def cond(carry):
        i, _, _, _, done = carry
        return jnp.logical_and(i < MAX_ITERS, jnp.min(done) < 0.5)

    def step(carry):
        i, lo, hi, thr, done = carry
        t = 0.5 * (lo + hi)
        cnt = count_gt(t)
        live = done < 0.5
        eq = jnp.logical_and(cnt == TOPK, live)
        thr = jnp.where(eq, t, thr)
        done = jnp.where(eq, 1.0, done)
        lo = jnp.where(jnp.logical_and(live, cnt >= TOPK), t, lo)
        hi = jnp.where(jnp.logical_and(live, cnt < TOPK), t, hi)
        return i + 1, lo, hi, thr, done

    _, lo, _, thr, done = lax.while_loop(
        cond, step, (jnp.int32(0), lo0, hi0, thr0, done0))
    thr = jnp.where(done > 0.5, thr, lo)

    masked = jnp.where(sim > thr, sim, 0.0)
    agg = lax.dot_general(masked, xa_ref[0],
                          (((1,), (0,)), ((), ())),
                          preferred_element_type=jnp.float32)

    o_ref[0] = lax.dot_general(agg, w_ref[...],
                               (((1,), (0,)), ((), ())),
                               preferred_element_type=jnp.float32) + b_ref[...]


def kernel(x, W, b):
    B, N, L = x.shape
    b2 = jnp.reshape(b, (1, L))
    grid = (B, N // BR)

    out = pl.pallas_call(
        _body,
        grid=grid,
        in_specs=[
            pl.BlockSpec((1, BR, L), lambda bb, j: (bb, j, 0)),
            pl.BlockSpec((1, N, L), lambda bb, j: (bb, 0, 0)),
            pl.BlockSpec((L, L), lambda bb, j: (0, 0)),
            pl.BlockSpec((1, L), lambda bb, j: (0, 0)),
        ],
        out_specs=pl.BlockSpec((1, BR, L), lambda bb, j: (bb, j, 0)),
        out_shape=jax.ShapeDtypeStruct((B, N, L), jnp.float32),
        scratch_shapes=[pltpu.VMEM((N, L), jnp.float32)],
        compiler_params=pltpu.CompilerParams(
            dimension_semantics=("arbitrary", "arbitrary"),
        ),
    )(x, x, W, b2)

    return (out, out)
